# baseline (speedup 1.0000x reference)
import functools

import jax
import jax.numpy as jnp
from jax import lax
from jax.experimental import pallas as pl
from jax.experimental.pallas import tpu as pltpu
from jax.experimental.pallas import tpu_sc as plsc

_NW = 32
_CHUNK = 1600
_NBUF = 2


def _sc_gather(table, flat_idx):
    btot = flat_idx.shape[0]
    d = table.shape[1]
    b_per_w = btot // _NW
    n_chunks = b_per_w // _CHUNK
    mesh = plsc.VectorSubcoreMesh(core_axis_name="c", subcore_axis_name="s")

    @functools.partial(
        pl.kernel,
        mesh=mesh,
        out_type=jax.ShapeDtypeStruct((btot, d), jnp.float32),
        compiler_params=pltpu.CompilerParams(use_tc_tiling_on_sc=False),
        scratch_types=[
            pltpu.VMEM((b_per_w,), jnp.int32),
            pltpu.VMEM((_CHUNK, d), jnp.float32),
            pltpu.VMEM((_CHUNK, d), jnp.float32),
            pltpu.SemaphoreType.DMA,
            pltpu.SemaphoreType.DMA,
            pltpu.SemaphoreType.DMA,
            pltpu.SemaphoreType.DMA,
        ],
    )
    def k(table_hbm, idx_hbm, out_hbm, idx_v, rows0, rows1, g0, g1, s0, s1):
        wid = lax.axis_index("s") * 2 + lax.axis_index("c")
        base = wid * b_per_w
        pltpu.sync_copy(idx_hbm.at[pl.ds(base, b_per_w)], idx_v)
        rows = (rows0, rows1)
        gsem = (g0, g1)
        ssem = (s0, s1)

        def gather_desc(c, b):
            src = table_hbm.at[idx_v.at[pl.ds(c * _CHUNK, _CHUNK)]]
            return pltpu.make_async_copy(src, rows[b], gsem[b])

        def store_desc(c, b):
            dst = out_hbm.at[pl.ds(base + c * _CHUNK, _CHUNK)]
            return pltpu.make_async_copy(rows[b], dst, ssem[b])

        for b in range(_NBUF):
            gather_desc(b, b).start()
        for b in range(_NBUF):
            gather_desc(b, b).wait()
            store_desc(b, b).start()

        @pl.loop(_NBUF, n_chunks, step=_NBUF)
        def body(g):
            for b in range(_NBUF):
                c = g + b
                store_desc(c - _NBUF, b).wait()
                gather_desc(c, b).start()
                gather_desc(c, b).wait()
                store_desc(c, b).start()

        for b in range(_NBUF):
            store_desc(n_chunks - _NBUF + b, b).wait()

    return k(table, flat_idx)


def _tc_transpose(x, n, h, d):
    x3 = x.reshape(h, n, d)

    def body(x_ref, o_ref):
        o_ref[0] = jnp.transpose(x_ref[0], (1, 0))

    return pl.pallas_call(
        body,
        grid=(h,),
        in_specs=[pl.BlockSpec((1, n, d), lambda i: (i, 0, 0))],
        out_specs=pl.BlockSpec((1, d, n), lambda i: (i, 0, 0)),
        out_shape=jax.ShapeDtypeStruct((h, d, n), jnp.float32),
    )(x3)


def kernel(indices, table):
    n, h = indices.shape
    d = table.shape[1]
    flat_idx = indices.T.reshape(-1)
    x = _sc_gather(table, flat_idx)
    out3 = _tc_transpose(x, n, h, d)
    return out3.transpose(2, 0, 1)
